# baseline (speedup 1.0000x reference)
import functools
import jax
import jax.numpy as jnp
from jax import lax
from jax.experimental import pallas as pl
from jax.experimental.pallas import tpu as pltpu
from jax.experimental.pallas import tpu_sc as plsc

N = 100000
F = 128
G = 512
EPS = 1e-05
NC = 2
NS = 16
NW = NC * NS

R = 160
NSUB = 5
SUB = 32
NBLK = N // R
JMAX = -(-NBLK // NW)

_MESH = plsc.VectorSubcoreMesh(
    core_axis_name="c", subcore_axis_name="s", num_cores=NC, num_subcores=NS
)


def _stats_sc(x_hbm, b_hbm, z128_hbm, z16_hbm, ones_hbm,
              psum_hbm, psq_hbm, pcnt_hbm,
              idx_v, xb_v, sq_v, ones_v, sp_sum, sp_sq, sp_cnt):
    c = lax.axis_index("c")
    s = lax.axis_index("s")
    wid = s * NC + c

    pltpu.sync_copy(z128_hbm.at[pl.ds(s * 32, 32)], sp_sum.at[pl.ds(s * 32, 32)])
    pltpu.sync_copy(z128_hbm.at[pl.ds(s * 32, 32)], sp_sq.at[pl.ds(s * 32, 32)])
    pltpu.sync_copy(z16_hbm.at[pl.ds(s * 32, 32)], sp_cnt.at[pl.ds(s * 32, 32)])
    pltpu.sync_copy(ones_hbm, ones_v)
    plsc.subcore_barrier()

    for j in range(JMAX):
        blk = wid + NW * j

        @pl.when(blk < NBLK)
        def _():
            base = blk * R
            pltpu.sync_copy(b_hbm.at[blk], idx_v)
            pltpu.sync_copy(x_hbm.at[pl.ds(base, R)], xb_v)

            def sq_row(r, carry):
                for jj in range(F // 16):
                    v = xb_v[r, pl.ds(jj * 16, 16)]
                    sq_v[r, pl.ds(jj * 16, 16)] = v * v
                return carry

            lax.fori_loop(0, R, sq_row, 0)

            for q in range(NSUB):
                rows = pl.ds(q * SUB, SUB)
                pltpu.sync_copy(xb_v.at[rows], sp_sum.at[idx_v.at[q]], add=True)
                pltpu.sync_copy(sq_v.at[rows], sp_sq.at[idx_v.at[q]], add=True)
                pltpu.sync_copy(ones_v.at[rows], sp_cnt.at[idx_v.at[q]], add=True)

    plsc.subcore_barrier()
    pltpu.sync_copy(sp_sum.at[pl.ds(s * 32, 32)], psum_hbm.at[c, pl.ds(s * 32, 32)])
    pltpu.sync_copy(sp_sq.at[pl.ds(s * 32, 32)], psq_hbm.at[c, pl.ds(s * 32, 32)])
    pltpu.sync_copy(sp_cnt.at[pl.ds(s * 32, 32)], pcnt_hbm.at[c, pl.ds(s * 32, 32)])


_stats_call = functools.partial(
    pl.kernel,
    out_type=[
        jax.ShapeDtypeStruct((NC, G, F), jnp.float32),
        jax.ShapeDtypeStruct((NC, G, F), jnp.float32),
        jax.ShapeDtypeStruct((NC, G, 16), jnp.float32),
    ],
    mesh=_MESH,
    scratch_types=[
        pltpu.VMEM((NSUB, SUB), jnp.int32),
        pltpu.VMEM((R, F), jnp.float32),
        pltpu.VMEM((R, F), jnp.float32),
        pltpu.VMEM((R, 16), jnp.float32),
        pltpu.VMEM_SHARED((G, F), jnp.float32),
        pltpu.VMEM_SHARED((G, F), jnp.float32),
        pltpu.VMEM_SHARED((G, 16), jnp.float32),
    ],
)(_stats_sc)


def _coef_tc(psum_ref, psq_ref, pcnt_ref, w_ref, b_ref, ms_ref, ab_ref):
    ssum = psum_ref[0] + psum_ref[1]
    ssq = psq_ref[0] + psq_ref[1]
    cnt = jnp.max(pcnt_ref[0] + pcnt_ref[1], axis=1, keepdims=True)
    c = jnp.maximum(cnt, 1.0)
    mean = ssum / c
    var = ssq / c - mean * mean
    rstd = lax.rsqrt(var + EPS)
    a = rstd * (ms_ref[...] * w_ref[...])
    bb = b_ref[...] - mean * a
    ab_ref[...] = jnp.concatenate([a, bb], axis=1)


def _apply_sc(x_hbm, b_hbm, ab_hbm, out_hbm, idx_v, xb_v, coef_v):
    c = lax.axis_index("c")
    s = lax.axis_index("s")
    wid = s * NC + c

    for j in range(JMAX):
        blk = wid + NW * j

        @pl.when(blk < NBLK)
        def _():
            base = blk * R
            pltpu.sync_copy(b_hbm.at[blk], idx_v)
            pltpu.sync_copy(x_hbm.at[pl.ds(base, R)], xb_v)
            for q in range(NSUB):
                rows = pl.ds(q * SUB, SUB)
                pltpu.sync_copy(ab_hbm.at[idx_v.at[q]], coef_v.at[rows])

            def fma_row(r, carry):
                for jj in range(F // 16):
                    v = xb_v[r, pl.ds(jj * 16, 16)]
                    a = coef_v[r, pl.ds(jj * 16, 16)]
                    b = coef_v[r, pl.ds(F + jj * 16, 16)]
                    xb_v[r, pl.ds(jj * 16, 16)] = v * a + b
                return carry

            lax.fori_loop(0, R, fma_row, 0)
            pltpu.sync_copy(xb_v, out_hbm.at[pl.ds(base, R)])


_apply_call = functools.partial(
    pl.kernel,
    out_type=jax.ShapeDtypeStruct((N, F), jnp.float32),
    mesh=_MESH,
    scratch_types=[
        pltpu.VMEM((NSUB, SUB), jnp.int32),
        pltpu.VMEM((R, F), jnp.float32),
        pltpu.VMEM((R, 2 * F), jnp.float32),
    ],
)(_apply_sc)


@jax.jit
def kernel(x, batch, weight, bias, mean_scale):
    b3 = batch.astype(jnp.int32).reshape(NBLK, NSUB, SUB)
    z128 = jnp.zeros((G, F), jnp.float32)
    z16 = jnp.zeros((G, 16), jnp.float32)
    ones_h = jnp.ones((R, 16), jnp.float32)

    psum, psq, pcnt = _stats_call(x, b3, z128, z16, ones_h)

    ab = pl.pallas_call(
        _coef_tc,
        in_specs=[
            pl.BlockSpec((NC, G, F), lambda: (0, 0, 0)),
            pl.BlockSpec((NC, G, F), lambda: (0, 0, 0)),
            pl.BlockSpec((NC, G, 16), lambda: (0, 0, 0)),
            pl.BlockSpec((1, F), lambda: (0, 0)),
            pl.BlockSpec((1, F), lambda: (0, 0)),
            pl.BlockSpec((1, F), lambda: (0, 0)),
        ],
        out_specs=pl.BlockSpec((G, 2 * F), lambda: (0, 0)),
        out_shape=jax.ShapeDtypeStruct((G, 2 * F), jnp.float32),
    )(psum, psq, pcnt, weight.reshape(1, F), bias.reshape(1, F),
      mean_scale.reshape(1, F))

    return _apply_call(x, b3, ab)


# speedup vs baseline: 2.6797x; 2.6797x over previous
import functools
import jax
import jax.numpy as jnp
from jax import lax
from jax.experimental import pallas as pl
from jax.experimental.pallas import tpu as pltpu
from jax.experimental.pallas import tpu_sc as plsc

N = 100000
F = 128
G = 512
EPS = 1e-05
NC = 2
NS = 16
NW = NC * NS

R = 160
NSUB = 5
SUB = 32
NBLK = N // R
JMAX = -(-NBLK // NW)

_MESH = plsc.VectorSubcoreMesh(
    core_axis_name="c", subcore_axis_name="s", num_cores=NC, num_subcores=NS
)


def _stats_sc(x_hbm, b_hbm, z128_hbm, z16_hbm, ones_hbm,
              psum_hbm, psq_hbm, pcnt_hbm,
              idx_v, xb_v, sq_v, ones_v, sp_sum, sp_sq, sp_cnt):
    c = lax.axis_index("c")
    s = lax.axis_index("s")
    wid = s * NC + c

    pltpu.sync_copy(z128_hbm.at[pl.ds(s * 32, 32)], sp_sum.at[pl.ds(s * 32, 32)])
    pltpu.sync_copy(z128_hbm.at[pl.ds(s * 32, 32)], sp_sq.at[pl.ds(s * 32, 32)])
    pltpu.sync_copy(z16_hbm.at[pl.ds(s * 32, 32)], sp_cnt.at[pl.ds(s * 32, 32)])
    pltpu.sync_copy(ones_hbm, ones_v)
    plsc.subcore_barrier()

    for j in range(JMAX):
        blk = wid + NW * j

        @pl.when(blk < NBLK)
        def _():
            base = blk * R
            pltpu.sync_copy(b_hbm.at[blk], idx_v)
            pltpu.sync_copy(x_hbm.at[pl.ds(base, R)], xb_v)

            def sq_row(r, carry):
                for jj in range(F // 16):
                    v = xb_v[r, pl.ds(jj * 16, 16)]
                    sq_v[r, pl.ds(jj * 16, 16)] = v * v
                return carry

            lax.fori_loop(0, R, sq_row, 0)

            for q in range(NSUB):
                rows = pl.ds(q * SUB, SUB)
                pltpu.sync_copy(xb_v.at[rows], sp_sum.at[idx_v.at[q]], add=True)
                pltpu.sync_copy(sq_v.at[rows], sp_sq.at[idx_v.at[q]], add=True)
                pltpu.sync_copy(ones_v.at[rows], sp_cnt.at[idx_v.at[q]], add=True)

    plsc.subcore_barrier()
    pltpu.sync_copy(sp_sum.at[pl.ds(s * 32, 32)], psum_hbm.at[c, pl.ds(s * 32, 32)])
    pltpu.sync_copy(sp_sq.at[pl.ds(s * 32, 32)], psq_hbm.at[c, pl.ds(s * 32, 32)])
    pltpu.sync_copy(sp_cnt.at[pl.ds(s * 32, 32)], pcnt_hbm.at[c, pl.ds(s * 32, 32)])


_stats_call = functools.partial(
    pl.kernel,
    out_type=[
        jax.ShapeDtypeStruct((NC, G, F), jnp.float32),
        jax.ShapeDtypeStruct((NC, G, F), jnp.float32),
        jax.ShapeDtypeStruct((NC, G, 16), jnp.float32),
    ],
    mesh=_MESH,
    scratch_types=[
        pltpu.VMEM((NSUB, SUB), jnp.int32),
        pltpu.VMEM((R, F), jnp.float32),
        pltpu.VMEM((R, F), jnp.float32),
        pltpu.VMEM((R, 16), jnp.float32),
        pltpu.VMEM_SHARED((G, F), jnp.float32),
        pltpu.VMEM_SHARED((G, F), jnp.float32),
        pltpu.VMEM_SHARED((G, 16), jnp.float32),
    ],
)(_stats_sc)


def _coef_tc(psum_ref, psq_ref, pcnt_ref, w_ref, b_ref, ms_ref, ab_ref):
    ssum = psum_ref[0] + psum_ref[1]
    ssq = psq_ref[0] + psq_ref[1]
    cnt = jnp.max(pcnt_ref[0] + pcnt_ref[1], axis=1, keepdims=True)
    c = jnp.maximum(cnt, 1.0)
    mean = ssum / c
    var = ssq / c - mean * mean
    rstd = lax.rsqrt(var + EPS)
    a = rstd * (ms_ref[...] * w_ref[...])
    bb = b_ref[...] - mean * a
    ab_ref[...] = jnp.concatenate([a, bb], axis=1)


def _apply_sc(x_hbm, b_hbm, a_hbm, bb_hbm, out_hbm,
              idx_v, xb_v, ca_v, cb_v, sp_a, sp_b, sem):
    c = lax.axis_index("c")
    s = lax.axis_index("s")
    wid = s * NC + c

    pltpu.sync_copy(a_hbm.at[pl.ds(s * 32, 32)], sp_a.at[pl.ds(s * 32, 32)])
    pltpu.sync_copy(bb_hbm.at[pl.ds(s * 32, 32)], sp_b.at[pl.ds(s * 32, 32)])
    plsc.subcore_barrier()

    for j in range(JMAX):
        blk = wid + NW * j

        @pl.when(blk < NBLK)
        def _():
            base = blk * R
            pltpu.sync_copy(b_hbm.at[blk], idx_v)
            descs = [
                pltpu.async_copy(
                    sp_a.at[idx_v.at[q]], ca_v.at[pl.ds(q * SUB, SUB)], sem
                )
                for q in range(NSUB)
            ] + [
                pltpu.async_copy(
                    sp_b.at[idx_v.at[q]], cb_v.at[pl.ds(q * SUB, SUB)], sem
                )
                for q in range(NSUB)
            ]
            pltpu.sync_copy(x_hbm.at[pl.ds(base, R)], xb_v)
            for d in descs:
                d.wait()

            def fma_row(r, carry):
                for jj in range(F // 16):
                    sl = pl.ds(jj * 16, 16)
                    xb_v[r, sl] = xb_v[r, sl] * ca_v[r, sl] + cb_v[r, sl]
                return carry

            lax.fori_loop(0, R, fma_row, 0)
            pltpu.sync_copy(xb_v, out_hbm.at[pl.ds(base, R)])


_apply_call = functools.partial(
    pl.kernel,
    out_type=jax.ShapeDtypeStruct((N, F), jnp.float32),
    mesh=_MESH,
    scratch_types=[
        pltpu.VMEM((NSUB, SUB), jnp.int32),
        pltpu.VMEM((R, F), jnp.float32),
        pltpu.VMEM((R, F), jnp.float32),
        pltpu.VMEM((R, F), jnp.float32),
        pltpu.VMEM_SHARED((G, F), jnp.float32),
        pltpu.VMEM_SHARED((G, F), jnp.float32),
        pltpu.SemaphoreType.DMA,
    ],
)(_apply_sc)


@jax.jit
def kernel(x, batch, weight, bias, mean_scale):
    b3 = batch.astype(jnp.int32).reshape(NBLK, NSUB, SUB)
    z128 = jnp.zeros((G, F), jnp.float32)
    z16 = jnp.zeros((G, 16), jnp.float32)
    ones_h = jnp.ones((R, 16), jnp.float32)

    psum, psq, pcnt = _stats_call(x, b3, z128, z16, ones_h)

    ab = pl.pallas_call(
        _coef_tc,
        in_specs=[
            pl.BlockSpec((NC, G, F), lambda: (0, 0, 0)),
            pl.BlockSpec((NC, G, F), lambda: (0, 0, 0)),
            pl.BlockSpec((NC, G, 16), lambda: (0, 0, 0)),
            pl.BlockSpec((1, F), lambda: (0, 0)),
            pl.BlockSpec((1, F), lambda: (0, 0)),
            pl.BlockSpec((1, F), lambda: (0, 0)),
        ],
        out_specs=pl.BlockSpec((G, 2 * F), lambda: (0, 0)),
        out_shape=jax.ShapeDtypeStruct((G, 2 * F), jnp.float32),
    )(psum, psq, pcnt, weight.reshape(1, F), bias.reshape(1, F),
      mean_scale.reshape(1, F))

    return _apply_call(x, b3, ab[:, :F], ab[:, F:])
